# baseline (speedup 1.0000x reference)
import functools

import jax
import jax.numpy as jnp
from jax import lax
from jax.experimental import pallas as pl
from jax.experimental.pallas import tpu as pltpu
from jax.experimental.pallas import tpu_sc as plsc

B = 4096
S = 13
E = 16
V = 100000
SE = S * E
N = B * S
NW = 32
PER_W = N // NW
BW = B // NW
NBLK = PER_W // 16

_sc_mesh = plsc.VectorSubcoreMesh(core_axis_name="c", subcore_axis_name="s")


@functools.partial(
    pl.kernel,
    mesh=_sc_mesh,
    compiler_params=pltpu.CompilerParams(use_tc_tiling_on_sc=False,
                                         needs_layout_passes=False),
    out_type=jax.ShapeDtypeStruct((N * E,), jnp.float32),
    scratch_types=[
        pltpu.VMEM((PER_W,), jnp.int32),
        pltpu.VMEM((S, BW), jnp.float32),
        pltpu.VMEM((PER_W * E,), jnp.int32),
        pltpu.VMEM((PER_W * E,), jnp.float32),
        pltpu.SemaphoreType.DMA,
    ],
)
def _sc_gather(idx_hbm, xvsp_hbm, t_hbm, out_hbm,
               idx_v, f_v, eidx, rows, sem):
    wid = lax.axis_index("s") * 2 + lax.axis_index("c")
    base = wid * PER_W
    pltpu.sync_copy(idx_hbm.at[pl.ds(base, PER_W)], idx_v)
    pltpu.sync_copy(xvsp_hbm.at[:, pl.ds(wid * BW, BW)], f_v)
    iota = lax.iota(jnp.int32, 16)

    def addr_body(j, _):
        ell = j * 16 + iota
        s = lax.rem(ell, S)
        v = plsc.load_gather(idx_v, [ell])
        base_e = s * (E * V) + v
        for e in range(E):
            plsc.store_scatter(eidx, [ell * E + e], base_e + e * V)
        return 0

    lax.fori_loop(0, NBLK, addr_body, 0)

    pltpu.async_copy(t_hbm.at[eidx], rows, sem).wait()

    def scale_body(j, _):
        ell = j * 16 + iota
        s = lax.rem(ell, S)
        brel = lax.div(ell, S)
        fvec = plsc.load_gather(f_v, [s, brel])
        for e in range(E):
            pos = ell * E + e
            r = plsc.load_gather(rows, [pos]) * fvec
            plsc.store_scatter(rows, [pos], r)
        return 0

    lax.fori_loop(0, NBLK, scale_body, 0)

    pltpu.sync_copy(rows, out_hbm.at[pl.ds(base * E, PER_W * E)])


def _tc_body(emb1, emb2, xirep, xils, wc1s, bc1s, xvrep,
             wc2f, bc2f, w1a, w1b, bl1, g1, be1, w2, bl2, g2, be2,
             m_eye, bias, out):
    f32 = jnp.float32
    fo = jnp.sum(emb1[...], axis=1, keepdims=True)
    t1 = xils[...] * wc1s[...] + bc1s[...]
    fo = fo + jnp.sum(t1 * xvrep[...], axis=1, keepdims=True)
    conv2 = xirep[...] * wc2f[...] + bc2f[...]
    fm2 = emb2[...]
    ssum = jnp.dot(conv2 + fm2, m_eye[...], preferred_element_type=f32)
    so = 0.5 * (jnp.sum(ssum * ssum, axis=1, keepdims=True)
                - jnp.sum(conv2 * conv2 + fm2 * fm2, axis=1, keepdims=True))
    z1 = (jnp.dot(conv2, w1a[...], preferred_element_type=f32)
          + jnp.dot(fm2, w1b[...], preferred_element_type=f32) + bl1[...])
    m1 = jnp.mean(z1, axis=0, keepdims=True)
    c1 = z1 - m1
    v1 = jnp.mean(c1 * c1, axis=0, keepdims=True)
    h1 = c1 * lax.rsqrt(v1 + 1e-5) * g1[...] + be1[...]
    z2 = jnp.dot(h1, w2[...], preferred_element_type=f32) + bl2[...]
    m2 = jnp.mean(z2, axis=0, keepdims=True)
    c2 = z2 - m2
    v2 = jnp.mean(c2 * c2, axis=0, keepdims=True)
    a2 = g2[...] * lax.rsqrt(v2 + 1e-5)
    hs = jnp.sum(c2 * a2, axis=1, keepdims=True) + jnp.sum(be2[...])
    out[...] = fo + so + hs + bias[...]


def kernel(Xi, Xv, Wc1, bc1, Wc2, bc2, E1t, E2t, Wl1, bl1, Wl2, bl2,
           g1, be1, g2, be2, bias):
    f32 = jnp.float32
    Xi_lin = Xi[:, :S, 0].astype(f32)
    idx_flat = Xi[:, S:, 0].reshape(-1)
    xvsp = Xv[:, S:].reshape(S, B)

    t1_flat = E1t.transpose(0, 2, 1).reshape(-1)
    emb1_f = _sc_gather(idx_flat, xvsp, t1_flat)
    t2_flat = E2t.transpose(0, 2, 1).reshape(-1)
    emb2_f = _sc_gather(idx_flat, xvsp, t2_flat)
    emb1 = emb1_f.reshape(B, SE)
    emb2 = emb2_f.reshape(B, SE)

    xirep = jnp.broadcast_to(Xi_lin[:, :, None], (B, S, E)).reshape(B, SE)
    xils = xirep.T.reshape(B, SE)
    wc1s = jnp.broadcast_to(Wc1.reshape(-1)[:, None], (SE, B)).reshape(B, SE)
    bc1s = jnp.broadcast_to(bc1.reshape(-1)[:, None], (SE, B)).reshape(B, SE)
    xvrep = jnp.broadcast_to(Xv[:, :S][:, :, None], (B, S, E)).reshape(B, SE)
    m_eye = jnp.tile(jnp.eye(E, dtype=f32), (S, 1))

    out = pl.pallas_call(
        _tc_body,
        out_shape=jax.ShapeDtypeStruct((B, 1), f32),
    )(emb1, emb2, xirep, xils, wc1s, bc1s, xvrep,
      Wc2.reshape(1, SE), bc2.reshape(1, SE),
      Wl1[:, :SE].T, Wl1[:, SE:].T, bl1.reshape(1, -1),
      g1.reshape(1, -1), be1.reshape(1, -1),
      Wl2.T, bl2.reshape(1, -1), g2.reshape(1, -1), be2.reshape(1, -1),
      m_eye, bias.reshape(B, 1))
    return out.reshape(B)
